# baseline (speedup 1.0000x reference)
import functools

import jax
import jax.numpy as jnp
from jax import lax
from jax.experimental import pallas as pl
from jax.experimental.pallas import tpu as pltpu
from jax.experimental.pallas import tpu_sc as plsc

B = 4096
H = 200
D = 32
GH = 100
NBUF = 8
RPI = 20
NACC = 4

_info = plsc.get_sparse_core_info()
NC, NS, L = _info.num_cores, _info.num_subcores, _info.num_lanes
NW = NC * NS
BPW = B // NW

_mesh = plsc.VectorSubcoreMesh(core_axis_name="c", subcore_axis_name="s")


@functools.partial(
    pl.kernel,
    mesh=_mesh,
    out_type=jax.ShapeDtypeStruct((B, D), jnp.float32),
    compiler_params=pltpu.CompilerParams(use_tc_tiling_on_sc=False),
    scratch_types=[
        pltpu.VMEM((2 * BPW, GH), jnp.int32),
        [pltpu.VMEM((H, L), jnp.int32) for _ in range(NBUF)],
        pltpu.VMEM((BPW, D), jnp.float32),
        [pltpu.SemaphoreType.DMA for _ in range(NBUF)],
    ],
)
def _encode(x_hbm, table_hbm, out_hbm, idx_v, rows, out_v, sems):
    wid = lax.axis_index("s") * NC + lax.axis_index("c")
    base = wid * BPW

    pltpu.sync_copy(x_hbm.at[pl.ds(2 * base, 2 * BPW)], idx_v)

    def start_gather(i, b):
        pltpu.async_copy(
            table_hbm.at[idx_v.at[2 * i]], rows[b].at[pl.ds(0, GH)], sems[b]
        )
        pltpu.async_copy(
            table_hbm.at[idx_v.at[2 * i + 1]], rows[b].at[pl.ds(GH, GH)], sems[b]
        )

    def wait_gather(i, b):
        pltpu.make_async_copy(
            table_hbm.at[idx_v.at[2 * i]], rows[b].at[pl.ds(0, GH)], sems[b]
        ).wait()
        pltpu.make_async_copy(
            table_hbm.at[idx_v.at[2 * i + 1]], rows[b].at[pl.ds(GH, GH)], sems[b]
        ).wait()

    def reduce_row(i, buf):
        zero = jnp.zeros((L,), jnp.float32)
        himask = jnp.full((L,), -65536, jnp.int32)

        def body(j, accs):
            accs = list(accs)
            for r in range(RPI):
                row = RPI * j + r
                even, odd = accs[r % NACC]
                w = buf[row, pl.ds(0, L)]
                even = even + lax.bitcast_convert_type(
                    lax.shift_left(w, 16), jnp.float32
                )
                odd = odd + lax.bitcast_convert_type(
                    lax.bitwise_and(w, himask), jnp.float32
                )
                accs[r % NACC] = (even, odd)
            return tuple(accs)

        accs = lax.fori_loop(0, H // RPI, body, tuple((zero, zero) for _ in range(NACC)))
        even = accs[0][0] + accs[1][0] + accs[2][0] + accs[3][0]
        odd = accs[0][1] + accs[1][1] + accs[2][1] + accs[3][1]
        scale = jnp.float32(1.0 / H)
        out_v[i, pl.ds(0, L)] = even * scale
        out_v[i, pl.ds(L, L)] = odd * scale

    for b in range(NBUF):
        start_gather(b, b)

    def outer(k, _):
        i0 = NBUF * k
        for b in range(NBUF):
            wait_gather(i0 + b, b)
            reduce_row(i0 + b, rows[b])
            start_gather(i0 + b + NBUF, b)
        return 0

    lax.fori_loop(0, BPW // NBUF - 1, outer, 0)

    for b in range(NBUF):
        i = BPW - NBUF + b
        wait_gather(i, b)
        reduce_row(i, rows[b])

    pltpu.sync_copy(out_v, out_hbm.at[pl.ds(base, BPW)])


def kernel(x, table):
    table_w = lax.bitcast_convert_type(
        table.astype(jnp.bfloat16).reshape(-1, L, 2), jnp.int32
    )
    halves = _encode(x.astype(jnp.int32).reshape(2 * B, GH), table_w)
    return jnp.stack([halves[:, :L], halves[:, L:]], axis=-1).reshape(B, D)


# speedup vs baseline: 2.0439x; 2.0439x over previous
import functools

import jax
import jax.numpy as jnp
from jax import lax
from jax.experimental import pallas as pl
from jax.experimental.pallas import tpu as pltpu
from jax.experimental.pallas import tpu_sc as plsc

B = 4096
H = 200
D = 32
GA = 104
GB = H - GA
NBUF = 8
RPI = 20
NACC = 4

_info = plsc.get_sparse_core_info()
NC, NS, L = _info.num_cores, _info.num_subcores, _info.num_lanes
NW = NC * NS
BPW = B // NW
IPW = BPW * H
OPW = BPW * D

_mesh = plsc.VectorSubcoreMesh(core_axis_name="c", subcore_axis_name="s")


@functools.partial(
    pl.kernel,
    mesh=_mesh,
    out_type=jax.ShapeDtypeStruct((B * D,), jnp.float32),
    compiler_params=pltpu.CompilerParams(use_tc_tiling_on_sc=False),
    scratch_types=[
        pltpu.VMEM((IPW,), jnp.int32),
        [pltpu.VMEM((H, D), jnp.float32) for _ in range(NBUF)],
        pltpu.VMEM((OPW,), jnp.float32),
        [pltpu.SemaphoreType.DMA for _ in range(NBUF)],
    ],
)
def _encode(x_hbm, table_hbm, out_hbm, idx_v, rows, out_v, sems):
    wid = lax.axis_index("s") * NC + lax.axis_index("c")

    pltpu.sync_copy(x_hbm.at[pl.ds(wid * IPW, IPW)], idx_v)

    def start_gather(i, b):
        pltpu.async_copy(
            table_hbm.at[idx_v.at[pl.ds(i * H, GA)]], rows[b].at[pl.ds(0, GA)], sems[b]
        )
        pltpu.async_copy(
            table_hbm.at[idx_v.at[pl.ds(i * H + GA, GB)]],
            rows[b].at[pl.ds(GA, GB)],
            sems[b],
        )

    def wait_gather(i, b):
        pltpu.make_async_copy(
            table_hbm.at[idx_v.at[pl.ds(i * H, GA)]], rows[b].at[pl.ds(0, GA)], sems[b]
        ).wait()
        pltpu.make_async_copy(
            table_hbm.at[idx_v.at[pl.ds(i * H + GA, GB)]],
            rows[b].at[pl.ds(GA, GB)],
            sems[b],
        ).wait()

    def reduce_row(i, buf):
        zero = jnp.zeros((L,), jnp.float32)

        def body(j, accs):
            accs = list(accs)
            for r in range(RPI):
                row = RPI * j + r
                lo, hi = accs[r % NACC]
                lo = lo + buf[row, pl.ds(0, L)]
                hi = hi + buf[row, pl.ds(L, L)]
                accs[r % NACC] = (lo, hi)
            return tuple(accs)

        accs = lax.fori_loop(0, H // RPI, body, tuple((zero, zero) for _ in range(NACC)))
        lo = accs[0][0] + accs[1][0] + accs[2][0] + accs[3][0]
        hi = accs[0][1] + accs[1][1] + accs[2][1] + accs[3][1]
        scale = jnp.float32(1.0 / H)
        out_v[pl.ds(i * D, L)] = lo * scale
        out_v[pl.ds(i * D + L, L)] = hi * scale

    for b in range(NBUF):
        start_gather(b, b)

    def outer(k, _):
        i0 = NBUF * k
        for b in range(NBUF):
            wait_gather(i0 + b, b)
            reduce_row(i0 + b, rows[b])
            start_gather(i0 + b + NBUF, b)
        return 0

    lax.fori_loop(0, BPW // NBUF - 1, outer, 0)

    for b in range(NBUF):
        i = BPW - NBUF + b
        wait_gather(i, b)
        reduce_row(i, rows[b])

    pltpu.sync_copy(out_v, out_hbm.at[pl.ds(wid * OPW, OPW)])


def kernel(x, table):
    flat = _encode(x.astype(jnp.int32).reshape(B * H), table)
    return flat.reshape(B, D)
